# baseline (speedup 1.0000x reference)
import functools

import jax
import jax.numpy as jnp
from jax.experimental import pallas as pl
from jax.experimental.pallas import tpu as pltpu

_C_PAD = 128


def _counts_body(thr_ref, x_ref, db_ref, lab_ref, out_ref, *, bn):
    i = pl.program_id(0)

    xb = jnp.where(x_ref[...] >= 0.0, 1.0, -1.0).astype(jnp.float8_e4m3fn)
    db = jnp.where(db_ref[...] >= 0.0, 1.0, -1.0).astype(jnp.float8_e4m3fn)

    dot = jax.lax.dot_general(
        xb, db, (((0,), (0,)), ((), ())),
        preferred_element_type=jnp.float32,
    )

    mask = (dot >= thr_ref[0]).astype(jnp.float8_e4m3fn)

    labs = lab_ref[0, 0, pl.ds(i * bn, bn)].reshape(1, bn)
    iota_c = jax.lax.broadcasted_iota(jnp.int32, (_C_PAD, bn), 0)
    oh_t = (labs == iota_c).astype(jnp.float8_e4m3fn)

    partial = jax.lax.dot_general(
        mask, oh_t, (((1,), (1,)), ((), ())),
        preferred_element_type=jnp.float32,
    )

    @pl.when(i == 0)
    def _init():
        out_ref[...] = partial

    @pl.when(i > 0)
    def _accum():
        out_ref[...] += partial


def _normalize_body(counts_ref, out_ref):
    counts = counts_ref[...]
    sums = jnp.sum(counts, axis=1, keepdims=True)
    probs = jnp.where(sums > 0.0, counts / jnp.maximum(sums, 1.0), 0.0)
    out_ref[...] = probs.T


def kernel(x, db_codes, db_labels, threshold):
    q, bits = x.shape
    n = db_codes.shape[0]

    bn = 4096
    nb = -(-n // bn)

    x_t = x.T
    db_t = db_codes.T

    labs_flat = jnp.pad(db_labels, (0, nb * bn - n), constant_values=-1)
    labs3 = labs_flat.reshape(1, 1, nb * bn)

    thr_dot = (jnp.asarray(bits, jnp.float32) - 2.0 * jnp.asarray(threshold, jnp.float32))
    thr_arr = thr_dot.reshape(1)

    counts = pl.pallas_call(
        functools.partial(_counts_body, bn=bn),
        grid=(nb,),
        in_specs=[
            pl.BlockSpec(memory_space=pltpu.SMEM),
            pl.BlockSpec((bits, q), lambda i: (0, 0)),
            pl.BlockSpec((bits, bn), lambda i: (0, i)),
            pl.BlockSpec((1, 1, nb * bn), lambda i: (0, 0, 0)),
        ],
        out_specs=pl.BlockSpec((q, _C_PAD), lambda i: (0, 0)),
        out_shape=jax.ShapeDtypeStruct((q, _C_PAD), jnp.float32),
        compiler_params=pltpu.CompilerParams(
            dimension_semantics=("arbitrary",),
        ),
    )(thr_arr, x_t, db_t, labs3)

    out = pl.pallas_call(
        _normalize_body,
        out_shape=jax.ShapeDtypeStruct((_C_PAD, q), jnp.float32),
    )(counts)

    return out[:100, :].T


# speedup vs baseline: 1.0567x; 1.0567x over previous
import functools

import jax
import jax.numpy as jnp
from jax.experimental import pallas as pl
from jax.experimental.pallas import tpu as pltpu

_C_PAD = 128


def _fused_body(thr_ref, x_ref, db_ref, lab_ref, out_ref, acc_ref, *, nb, bn):
    i = pl.program_id(0)

    xb = jnp.where(x_ref[...] >= 0.0, 1.0, -1.0).astype(jnp.float8_e4m3fn)
    db = jnp.where(db_ref[...] >= 0.0, 1.0, -1.0).astype(jnp.float8_e4m3fn)

    dot = jax.lax.dot_general(
        xb, db, (((0,), (0,)), ((), ())),
        preferred_element_type=jnp.float32,
    )

    mask = (dot >= thr_ref[0]).astype(jnp.float8_e4m3fn)

    labs = lab_ref[0, 0, pl.ds(i * bn, bn)].reshape(1, bn)
    iota_c = jax.lax.broadcasted_iota(jnp.int32, (_C_PAD, bn), 0)
    oh_t = (labs == iota_c).astype(jnp.float8_e4m3fn)

    partial = jax.lax.dot_general(
        mask, oh_t, (((1,), (1,)), ((), ())),
        preferred_element_type=jnp.float32,
    )

    @pl.when(i == 0)
    def _init():
        acc_ref[...] = partial

    @pl.when(i > 0)
    def _accum():
        acc_ref[...] += partial

    @pl.when(i == nb - 1)
    def _finish():
        counts = acc_ref[...]
        sums = jnp.sum(counts, axis=1, keepdims=True)
        probs = jnp.where(sums > 0.0, counts / jnp.maximum(sums, 1.0), 0.0)
        out_ref[...] = probs.T


def kernel(x, db_codes, db_labels, threshold):
    q, bits = x.shape
    n = db_codes.shape[0]

    bn = 8192
    nb = -(-n // bn)

    x_t = x.T
    db_t = db_codes.T

    labs_flat = jnp.pad(db_labels, (0, nb * bn - n), constant_values=-1)
    labs3 = labs_flat.reshape(1, 1, nb * bn)

    thr_dot = (jnp.asarray(bits, jnp.float32) - 2.0 * jnp.asarray(threshold, jnp.float32))
    thr_arr = thr_dot.reshape(1)

    out = pl.pallas_call(
        functools.partial(_fused_body, nb=nb, bn=bn),
        grid=(nb,),
        in_specs=[
            pl.BlockSpec(memory_space=pltpu.SMEM),
            pl.BlockSpec((bits, q), lambda i: (0, 0)),
            pl.BlockSpec((bits, bn), lambda i: (0, i)),
            pl.BlockSpec((1, 1, nb * bn), lambda i: (0, 0, 0)),
        ],
        out_specs=pl.BlockSpec((_C_PAD, q), lambda i: (0, 0)),
        out_shape=jax.ShapeDtypeStruct((_C_PAD, q), jnp.float32),
        scratch_shapes=[pltpu.VMEM((q, _C_PAD), jnp.float32)],
        compiler_params=pltpu.CompilerParams(
            dimension_semantics=("arbitrary",),
        ),
    )(thr_arr, x_t, db_t, labs3)

    return out[:100, :].T


# speedup vs baseline: 1.0764x; 1.0187x over previous
import functools

import jax
import jax.numpy as jnp
from jax.experimental import pallas as pl
from jax.experimental.pallas import tpu as pltpu

_C_PAD = 128


def _fused_body(thr_ref, x_ref, db_ref, lab_ref, out_ref, acc_ref, *, nb, bn):
    i = pl.program_id(0)

    xb = jnp.where(x_ref[...] >= 0.0, 1.0, -1.0).astype(jnp.float8_e4m3fn)
    db = jnp.where(db_ref[...] >= 0.0, 1.0, -1.0).astype(jnp.float8_e4m3fn)

    dot = jax.lax.dot_general(
        xb, db, (((0,), (0,)), ((), ())),
        preferred_element_type=jnp.float32,
    )

    mask = (dot >= thr_ref[0]).astype(jnp.float8_e4m3fn)

    labs = lab_ref[0, 0, pl.ds(i * bn, bn)].reshape(1, bn)
    iota_c = jax.lax.broadcasted_iota(jnp.int32, (_C_PAD, bn), 0)
    oh_t = (labs == iota_c).astype(jnp.float8_e4m3fn)

    partial = jax.lax.dot_general(
        mask, oh_t, (((1,), (1,)), ((), ())),
        preferred_element_type=jnp.float32,
    )

    @pl.when(i == 0)
    def _init():
        acc_ref[...] = partial

    @pl.when(i > 0)
    def _accum():
        acc_ref[...] += partial

    @pl.when(i == nb - 1)
    def _finish():
        counts = acc_ref[...]
        sums = jnp.sum(counts, axis=1, keepdims=True)
        probs = jnp.where(sums > 0.0, counts / jnp.maximum(sums, 1.0), 0.0)
        out_ref[...] = probs.T


def kernel(x, db_codes, db_labels, threshold):
    q, bits = x.shape
    n = db_codes.shape[0]

    bn = 6400
    nb = -(-n // bn)

    x_t = x.T
    db_t = db_codes.T

    labs_flat = jnp.pad(db_labels, (0, nb * bn - n), constant_values=-1)
    labs3 = labs_flat.reshape(1, 1, nb * bn)

    thr_dot = (jnp.asarray(bits, jnp.float32) - 2.0 * jnp.asarray(threshold, jnp.float32))
    thr_arr = thr_dot.reshape(1)

    out = pl.pallas_call(
        functools.partial(_fused_body, nb=nb, bn=bn),
        grid=(nb,),
        in_specs=[
            pl.BlockSpec(memory_space=pltpu.SMEM),
            pl.BlockSpec((bits, q), lambda i: (0, 0)),
            pl.BlockSpec((bits, bn), lambda i: (0, i)),
            pl.BlockSpec((1, 1, nb * bn), lambda i: (0, 0, 0)),
        ],
        out_specs=pl.BlockSpec((_C_PAD, q), lambda i: (0, 0)),
        out_shape=jax.ShapeDtypeStruct((_C_PAD, q), jnp.float32),
        scratch_shapes=[pltpu.VMEM((q, _C_PAD), jnp.float32)],
        compiler_params=pltpu.CompilerParams(
            dimension_semantics=("arbitrary",),
        ),
    )(thr_arr, x_t, db_t, labs3)

    return out[:100, :].T


# speedup vs baseline: 1.1103x; 1.0314x over previous
import functools

import jax
import jax.numpy as jnp
from jax.experimental import pallas as pl
from jax.experimental.pallas import tpu as pltpu

_C_PAD = 128


def _fused_body(thr_ref, x_ref, db_ref, lab_ref, out_ref, acc_ref, *, nb, bn):
    i = pl.program_id(0)

    xb = jnp.where(x_ref[...] >= 0.0, 1.0, -1.0).astype(jnp.float8_e4m3fn)
    db = jnp.where(db_ref[...] >= 0.0, 1.0, -1.0).astype(jnp.float8_e4m3fn)

    dot = jax.lax.dot_general(
        xb, db, (((0,), (0,)), ((), ())),
        preferred_element_type=jnp.float32,
    )

    mask = (dot >= thr_ref[0]).astype(jnp.float8_e4m3fn)

    labs = lab_ref[0, 0, pl.ds(i * bn, bn)].reshape(1, bn)
    iota_c = jax.lax.broadcasted_iota(jnp.int32, (_C_PAD, bn), 0)
    oh_t = (labs == iota_c).astype(jnp.float8_e4m3fn)

    partial = jax.lax.dot_general(
        mask, oh_t, (((1,), (1,)), ((), ())),
        preferred_element_type=jnp.float32,
    )

    @pl.when(i == 0)
    def _init():
        acc_ref[...] = partial

    @pl.when(i > 0)
    def _accum():
        acc_ref[...] += partial

    @pl.when(i == nb - 1)
    def _finish():
        counts = acc_ref[...]
        sums = jnp.sum(counts, axis=1, keepdims=True)
        probs = jnp.where(sums > 0.0, counts / jnp.maximum(sums, 1.0), 0.0)
        out_ref[...] = probs.T


def kernel(x, db_codes, db_labels, threshold):
    q, bits = x.shape
    n = db_codes.shape[0]

    bn = 10240
    nb = -(-n // bn)

    x_t = x.T
    db_t = db_codes.T

    labs_flat = jnp.pad(db_labels, (0, nb * bn - n), constant_values=-1)
    labs3 = labs_flat.reshape(1, 1, nb * bn)

    thr_dot = (jnp.asarray(bits, jnp.float32) - 2.0 * jnp.asarray(threshold, jnp.float32))
    thr_arr = thr_dot.reshape(1)

    out = pl.pallas_call(
        functools.partial(_fused_body, nb=nb, bn=bn),
        grid=(nb,),
        in_specs=[
            pl.BlockSpec(memory_space=pltpu.SMEM),
            pl.BlockSpec((bits, q), lambda i: (0, 0)),
            pl.BlockSpec((bits, bn), lambda i: (0, i)),
            pl.BlockSpec((1, 1, nb * bn), lambda i: (0, 0, 0)),
        ],
        out_specs=pl.BlockSpec((_C_PAD, q), lambda i: (0, 0)),
        out_shape=jax.ShapeDtypeStruct((_C_PAD, q), jnp.float32),
        scratch_shapes=[pltpu.VMEM((q, _C_PAD), jnp.float32)],
        compiler_params=pltpu.CompilerParams(
            dimension_semantics=("arbitrary",),
        ),
    )(thr_arr, x_t, db_t, labs3)

    return out[:100, :].T


# speedup vs baseline: 1.1365x; 1.0236x over previous
import functools

import jax
import jax.numpy as jnp
from jax.experimental import pallas as pl
from jax.experimental.pallas import tpu as pltpu

_C_PAD = 128


def _fused_body(thr_ref, x_ref, db_ref, lab_ref, out_ref, acc_ref, *, nb, bn):
    i = pl.program_id(0)

    xb = jnp.where(x_ref[...] >= 0.0, 1.0, -1.0).astype(jnp.float8_e4m3fn)
    db = jnp.where(db_ref[...] >= 0.0, 1.0, -1.0).astype(jnp.float8_e4m3fn)

    dot = jax.lax.dot_general(
        xb, db, (((0,), (0,)), ((), ())),
        preferred_element_type=jnp.float32,
    )

    mask = (dot >= thr_ref[0]).astype(jnp.float8_e4m3fn)

    labs = lab_ref[0, 0, pl.ds(i * bn, bn)].reshape(1, bn)
    iota_c = jax.lax.broadcasted_iota(jnp.int32, (_C_PAD, bn), 0)
    oh_t = (labs == iota_c).astype(jnp.float8_e4m3fn)

    partial = jax.lax.dot_general(
        mask, oh_t, (((1,), (1,)), ((), ())),
        preferred_element_type=jnp.float32,
    )

    @pl.when(i == 0)
    def _init():
        acc_ref[...] = partial

    @pl.when(i > 0)
    def _accum():
        acc_ref[...] += partial

    @pl.when(i == nb - 1)
    def _finish():
        counts = acc_ref[...]
        sums = jnp.sum(counts, axis=1, keepdims=True)
        probs = jnp.where(sums > 0.0, counts / jnp.maximum(sums, 1.0), 0.0)
        out_ref[...] = probs.T


def kernel(x, db_codes, db_labels, threshold):
    q, bits = x.shape
    n = db_codes.shape[0]

    bn = 12544
    nb = -(-n // bn)

    x_t = x.T
    db_t = db_codes.T

    labs_flat = jnp.pad(db_labels, (0, nb * bn - n), constant_values=-1)
    labs3 = labs_flat.reshape(1, 1, nb * bn)

    thr_dot = (jnp.asarray(bits, jnp.float32) - 2.0 * jnp.asarray(threshold, jnp.float32))
    thr_arr = thr_dot.reshape(1)

    out = pl.pallas_call(
        functools.partial(_fused_body, nb=nb, bn=bn),
        grid=(nb,),
        in_specs=[
            pl.BlockSpec(memory_space=pltpu.SMEM),
            pl.BlockSpec((bits, q), lambda i: (0, 0)),
            pl.BlockSpec((bits, bn), lambda i: (0, i)),
            pl.BlockSpec((1, 1, nb * bn), lambda i: (0, 0, 0)),
        ],
        out_specs=pl.BlockSpec((_C_PAD, q), lambda i: (0, 0)),
        out_shape=jax.ShapeDtypeStruct((_C_PAD, q), jnp.float32),
        scratch_shapes=[pltpu.VMEM((q, _C_PAD), jnp.float32)],
        compiler_params=pltpu.CompilerParams(
            dimension_semantics=("arbitrary",),
        ),
    )(thr_arr, x_t, db_t, labs3)

    return out[:100, :].T
